# Initial kernel scaffold; baseline (speedup 1.0000x reference)
import jax
import jax.numpy as jnp
from jax.experimental import pallas as pl


def kernel(x, observations):
    raise NotImplementedError("write your pallas kernel here")


# speedup vs baseline: 2.4319x; 2.4319x over previous
import functools

import jax
import jax.numpy as jnp
from jax.experimental import pallas as pl
from jax.experimental.pallas import tpu as pltpu

_TK = 2048
_QC = 8


def _nn_tile_kernel(nq, tk, nqc, xt_ref, obst_ref, min_ref, idx_ref,
                    dist_scratch, absk_scratch):
    t = pl.program_id(0)
    qq = pl.program_id(1)

    @pl.when(qq == 0)
    def _():
        absk_scratch[...] = jnp.abs(obst_ref[...])

    keys = obst_ref[...]
    absk = absk_scratch[...]
    xq8 = xt_ref[0]
    axq8 = jnp.maximum(jnp.abs(xq8), 1e-30)

    rows = []
    for j in range(_QC):
        xq = jax.lax.slice(xq8, (0, j), (xq8.shape[0], j + 1))
        axq = jax.lax.slice(axq8, (0, j), (axq8.shape[0], j + 1))
        num = jnp.abs(keys - xq)
        den = absk + axq
        rows.append(jnp.sum(num / den, axis=0, keepdims=True))
    dist8 = jnp.concatenate(rows, axis=0)
    dist_scratch[pl.ds(qq * _QC, _QC), :] = dist8

    @pl.when(qq == nqc - 1)
    def _():
        dmat = dist_scratch[...]
        m = jnp.min(dmat, axis=1, keepdims=True)
        am = jnp.argmin(dmat, axis=1).astype(jnp.int32)[:, None]
        gi = am + t * tk

        @pl.when(t == 0)
        def _():
            min_ref[...] = m
            idx_ref[...] = gi

        @pl.when(t > 0)
        def _():
            old_m = min_ref[...]
            take = m < old_m
            min_ref[...] = jnp.where(take, m, old_m)
            idx_ref[...] = jnp.where(take, gi, idx_ref[...])


def kernel(x, observations):
    nq, d = x.shape
    k = observations.shape[0]
    tk = _TK
    ntiles = -(-k // tk)
    kpad = ntiles * tk
    nqc = nq // _QC

    obst = observations.T
    if kpad > k:
        pad = jnp.broadcast_to(obst[:, :1], (d, kpad - k))
        obst = jnp.concatenate([obst, pad], axis=1)
    xt3 = x.reshape(nqc, _QC, d).transpose(0, 2, 1)

    min2d, idx2d = pl.pallas_call(
        functools.partial(_nn_tile_kernel, nq, tk, nqc),
        grid=(ntiles, nqc),
        in_specs=[
            pl.BlockSpec((1, d, _QC), lambda t, q: (q, 0, 0)),
            pl.BlockSpec((d, tk), lambda t, q: (0, t)),
        ],
        out_specs=[
            pl.BlockSpec((nq, 1), lambda t, q: (0, 0)),
            pl.BlockSpec((nq, 1), lambda t, q: (0, 0)),
        ],
        out_shape=[
            jax.ShapeDtypeStruct((nq, 1), jnp.float32),
            jax.ShapeDtypeStruct((nq, 1), jnp.int32),
        ],
        scratch_shapes=[
            pltpu.VMEM((nq, tk), jnp.float32),
            pltpu.VMEM((d, tk), jnp.float32),
        ],
    )(xt3, obst)

    return min2d[:, 0], idx2d[:, 0]
